# Initial kernel scaffold; baseline (speedup 1.0000x reference)
import jax
import jax.numpy as jnp
from jax.experimental import pallas as pl


def kernel(input_labels, pos_labels, neg_labels, center_embedding, back_embedding):
    raise NotImplementedError("write your pallas kernel here")


# speedup vs baseline: 1.6256x; 1.6256x over previous
import functools

import jax
import jax.numpy as jnp
from jax import lax
from jax.experimental import pallas as pl
from jax.experimental.pallas import tpu as pltpu
from jax.experimental.pallas import tpu_sc as plsc

VOCAB = 1000000
EMBED = 64
BATCH = 16384
P = 4
N = 20
ROWS = P + N

NUM_WORKERS = 32
PER_W = BATCH // NUM_WORKERS
CHUNK = 32
NCHUNK = PER_W // CHUNK
CTX = CHUNK * ROWS
GSLICE = 128
NG = CTX // GSLICE


def _sc_dots(input_labels, pos_flat, neg_flat, center, back):
    mesh = plsc.VectorSubcoreMesh(
        core_axis_name="c", subcore_axis_name="s", num_cores=2, num_subcores=16)

    @functools.partial(
        pl.kernel,
        mesh=mesh,
        out_type=jax.ShapeDtypeStruct((BATCH, 32), jnp.float32),
        compiler_params=pltpu.CompilerParams(
            needs_layout_passes=False, use_tc_tiling_on_sc=False),
        scratch_types=[
            pltpu.VMEM((CHUNK,), jnp.int32),
            pltpu.VMEM((CTX,), jnp.int32),
            pltpu.VMEM((CHUNK, EMBED), jnp.float32),
            pltpu.VMEM((CTX, EMBED), jnp.float32),
            pltpu.VMEM((CHUNK, 32), jnp.float32),
            pltpu.SemaphoreType.DMA,
            pltpu.SemaphoreType.DMA,
        ],
    )
    def k(inlab_hbm, pos_hbm, neg_hbm, center_hbm, back_hbm, out_hbm,
          idxc_v, idxb_v, in_rows, ctx_rows, dots_v, sem_c, sem_b):
        wid = lax.axis_index("s") * 2 + lax.axis_index("c")
        lane = lax.iota(jnp.int32, 16)

        @pl.loop(0, NCHUNK)
        def _chunk(kk):
            base = wid * PER_W + kk * CHUNK
            pltpu.sync_copy(inlab_hbm.at[pl.ds(base, CHUNK)], idxc_v)
            pltpu.sync_copy(pos_hbm.at[pl.ds(base * P, CHUNK * P)],
                            idxb_v.at[pl.ds(0, CHUNK * P)])
            pltpu.sync_copy(neg_hbm.at[pl.ds(base * N, CHUNK * N)],
                            idxb_v.at[pl.ds(CHUNK * P, CHUNK * N)])
            cdesc = pltpu.async_copy(center_hbm.at[idxc_v], in_rows, sem_c)
            gds = []
            for j in range(NG):
                gds.append(pltpu.async_copy(
                    back_hbm.at[idxb_v.at[pl.ds(j * GSLICE, GSLICE)]],
                    ctx_rows.at[pl.ds(j * GSLICE, GSLICE)],
                    sem_b))
            cdesc.wait()
            for d in gds:
                d.wait()

            @pl.loop(0, CHUNK)
            def _elem(b):
                ins = [in_rows[b, pl.ds(16 * q, 16)] for q in range(4)]
                lo = jnp.zeros((16,), jnp.float32)
                hi = jnp.zeros((16,), jnp.float32)
                for r in range(ROWS):
                    if r < P:
                        ro = P * b + r
                    else:
                        ro = CHUNK * P + N * b + (r - P)
                    acc = ins[0] * ctx_rows[ro, pl.ds(0, 16)]
                    for q in range(1, 4):
                        acc = acc + ins[q] * ctx_rows[ro, pl.ds(16 * q, 16)]
                    d = jnp.sum(acc)
                    if r < 16:
                        lo = jnp.where(lane == r, d, lo)
                    else:
                        hi = jnp.where(lane == (r - 16), d, hi)
                dots_v[b, pl.ds(0, 16)] = lo
                dots_v[b, pl.ds(16, 16)] = hi

            pltpu.sync_copy(dots_v, out_hbm.at[pl.ds(base, CHUNK)])

    return k(input_labels, pos_flat, neg_flat, center, back)


def _logsig(x):
    return jnp.minimum(x, 0.0) - jnp.log1p(jnp.exp(-jnp.abs(x)))


def _loss_body(d_ref, o_ref):
    x = d_ref[...]
    pos = x[:, 0:P]
    neg = x[:, P:ROWS]
    lp = jnp.sum(_logsig(pos), axis=1)
    ln = jnp.sum(_logsig(-neg), axis=1)
    o_ref[...] = -(lp + ln)


def _tc_loss(dots):
    blk = 1024
    return pl.pallas_call(
        _loss_body,
        grid=(BATCH // blk,),
        in_specs=[pl.BlockSpec((blk, 32), lambda i: (i, 0))],
        out_specs=pl.BlockSpec((blk,), lambda i: (i,)),
        out_shape=jax.ShapeDtypeStruct((BATCH,), jnp.float32),
    )(dots)


def kernel(input_labels, pos_labels, neg_labels, center_embedding, back_embedding):
    inlab = input_labels.astype(jnp.int32)
    pos_flat = pos_labels.astype(jnp.int32).reshape(-1)
    neg_flat = neg_labels.astype(jnp.int32).reshape(-1)
    dots = _sc_dots(inlab, pos_flat, neg_flat, center_embedding, back_embedding)
    return _tc_loss(dots)
